# Initial kernel scaffold; baseline (speedup 1.0000x reference)
import jax
import jax.numpy as jnp
from jax.experimental import pallas as pl


def kernel(x, edge_attr, edge_nei, params, wyckoff, inv_edge_type, edge_index, inv_edge_index, batch):
    raise NotImplementedError("write your pallas kernel here")


# speedup vs baseline: 1.3786x; 1.3786x over previous
import functools
import math

import jax
import jax.numpy as jnp
from jax import lax
from jax.experimental import pallas as pl
from jax.experimental.pallas import tpu as pltpu
from jax.experimental.pallas import tpu_sc as plsc

F32 = jnp.float32
N_NODES = 10000
E_REG = 160000
N_INV = 40000
EA = E_REG + N_INV
EA_PAD = 204800
BINS = 256
NODE_F = 256
N_GRAPHS = 64
INV_SQRT = 1.0 / math.sqrt(NODE_F)

_INTERPRET = False


def _rbf_body(gamma, vmin, step, d_ref, g_ref, w_ref, br_ref, u_ref, cv_ref,
              o_ref):
    d = d_ref[...]
    g = g_ref[...]
    centers = vmin + step * lax.broadcasted_iota(jnp.int32, (1, BINS),
                                                 1).astype(F32)
    feat = jnp.exp(-gamma * (d - centers) ** 2)
    acc = jnp.dot(feat, w_ref[...], preferred_element_type=F32)
    acc = acc + br_ref[...] + (d * g) * u_ref[...] + g * cv_ref[...]
    o_ref[...] = jax.nn.softplus(acc)


def _rbf_block(d, g, w, br, u, cv, vmin, vmax, Bb):
    R = d.shape[0]
    step = (vmax - vmin) / (BINS - 1)
    gamma = 1.0 / step
    return pl.pallas_call(
        functools.partial(_rbf_body, gamma, vmin, step),
        grid=(R // Bb,),
        in_specs=[
            pl.BlockSpec((Bb, 1), lambda i: (i, 0)),
            pl.BlockSpec((Bb, 1), lambda i: (i, 0)),
            pl.BlockSpec((BINS, NODE_F), lambda i: (0, 0)),
            pl.BlockSpec((1, NODE_F), lambda i: (0, 0)),
            pl.BlockSpec((1, NODE_F), lambda i: (0, 0)),
            pl.BlockSpec((1, NODE_F), lambda i: (0, 0)),
        ],
        out_specs=pl.BlockSpec((Bb, NODE_F), lambda i: (i, 0)),
        out_shape=jax.ShapeDtypeStruct((R, NODE_F), F32),
        interpret=_INTERPRET,
    )(d, g, w, br, u, cv)


def _mm_body(x_ref, w_ref, o_ref):
    o_ref[...] = jnp.dot(x_ref[...], w_ref[...], preferred_element_type=F32)


def _mm(x, w, Bb, rows_out=None):
    R = rows_out if rows_out is not None else x.shape[0]
    K = x.shape[1]
    Nw = w.shape[1]
    return pl.pallas_call(
        _mm_body,
        grid=(R // Bb,),
        in_specs=[
            pl.BlockSpec((Bb, K), lambda i: (i, 0)),
            pl.BlockSpec((K, Nw), lambda i: (0, 0)),
        ],
        out_specs=pl.BlockSpec((Bb, Nw), lambda i: (i, 0)),
        out_shape=jax.ShapeDtypeStruct((R, Nw), F32),
        interpret=_INTERPRET,
    )(x, w)


def _nf_embed_body(x_ref, wy_ref, wx_ref, ew_ref, b_ref, o_ref):
    oh = (wy_ref[...] == lax.broadcasted_iota(jnp.int32, (x_ref.shape[0], 128), 1)
          ).astype(F32)
    acc = jnp.dot(x_ref[...], wx_ref[...], preferred_element_type=F32)
    acc = acc + jnp.dot(oh, ew_ref[...], preferred_element_type=F32)
    o_ref[...] = acc + b_ref[...]


def _nf_embed(xpad, wy, wxp, ewp, b):
    Bb = 1000
    return pl.pallas_call(
        _nf_embed_body,
        grid=(N_NODES // Bb,),
        in_specs=[
            pl.BlockSpec((Bb, 128), lambda i: (i, 0)),
            pl.BlockSpec((Bb, 1), lambda i: (i, 0)),
            pl.BlockSpec((128, NODE_F), lambda i: (0, 0)),
            pl.BlockSpec((128, NODE_F), lambda i: (0, 0)),
            pl.BlockSpec((1, NODE_F), lambda i: (0, 0)),
        ],
        out_specs=pl.BlockSpec((Bb, NODE_F), lambda i: (i, 0)),
        out_shape=jax.ShapeDtypeStruct((N_NODES, NODE_F), F32),
        interpret=_INTERPRET,
    )(xpad, wy, wxp, ewp, b)


def _passA_body(Bb, qg_ref, kd_ref, ks_ref, ek_ref, w1_ref, bq_ref, b0_ref,
                b1_ref, a_ref, s1_ref, s2_ref):
    i = pl.program_id(0)
    hk = jax.nn.silu(kd_ref[...] + ks_ref[...] + ek_ref[...] + b0_ref[...])
    kj = jnp.dot(hk, w1_ref[...], preferred_element_type=F32) + b1_ref[...]
    alpha = (qg_ref[...] + bq_ref[...]) * kj * INV_SQRT
    a_ref[...] = alpha
    gidx = i * Bb + lax.broadcasted_iota(jnp.int32, (Bb, 1), 0)
    am = jnp.where(gidx < EA, alpha, 0.0)

    @pl.when(i == 0)
    def _():
        s1_ref[...] = jnp.zeros_like(s1_ref)
        s2_ref[...] = jnp.zeros_like(s2_ref)

    s1_ref[...] += jnp.broadcast_to(jnp.sum(am, 0, keepdims=True), (8, NODE_F))
    s2_ref[...] += jnp.broadcast_to(jnp.sum(am * am, 0, keepdims=True),
                                    (8, NODE_F))


def _passA(gTD, gTS, EF2, w1k, bq, b0k, b1k):
    Bb = 1024
    grid = EA_PAD // Bb
    return pl.pallas_call(
        functools.partial(_passA_body, Bb),
        grid=(grid,),
        in_specs=[
            pl.BlockSpec((Bb, NODE_F), lambda i: (i, 0)),
            pl.BlockSpec((Bb, NODE_F), lambda i: (i, 1)),
            pl.BlockSpec((Bb, NODE_F), lambda i: (i, 0)),
            pl.BlockSpec((Bb, NODE_F), lambda i: (i, 0)),
            pl.BlockSpec((NODE_F, NODE_F), lambda i: (0, 0)),
            pl.BlockSpec((1, NODE_F), lambda i: (0, 0)),
            pl.BlockSpec((1, NODE_F), lambda i: (0, 0)),
            pl.BlockSpec((1, NODE_F), lambda i: (0, 0)),
        ],
        out_specs=[
            pl.BlockSpec((Bb, NODE_F), lambda i: (i, 0)),
            pl.BlockSpec((8, NODE_F), lambda i: (0, 0)),
            pl.BlockSpec((8, NODE_F), lambda i: (0, 0)),
        ],
        out_shape=[
            jax.ShapeDtypeStruct((EA_PAD, NODE_F), F32),
            jax.ShapeDtypeStruct((8, NODE_F), F32),
            jax.ShapeDtypeStruct((8, NODE_F), F32),
        ],
        interpret=_INTERPRET,
    )(gTD, gTD, gTS, EF2, w1k, bq, b0k, b1k)


def _passB_body(Bb, vd_ref, vs_ref, em_ref, a_ref, sc_ref, sh_ref, w1_ref,
                b0_ref, b1_ref, ma_ref, mb_ref):
    i = pl.program_id(0)
    hm = jax.nn.silu(vd_ref[...] + vs_ref[...] + em_ref[...] + b0_ref[...])
    msg = jnp.dot(hm, w1_ref[...], preferred_element_type=F32) + b1_ref[...]
    gate = jax.nn.sigmoid(a_ref[...] * sc_ref[...] + sh_ref[...])
    mf = msg * gate
    gidx = i * Bb + lax.broadcasted_iota(jnp.int32, (Bb, 1), 0)
    mf = jnp.where(gidx < EA, mf, 0.0)
    ma_ref[...] = mf[:, :128]
    mb_ref[...] = mf[:, 128:]


def _passB(gTD, gTS, EF2, alpha, scale, shift, w1m, b0m, b1m):
    Bb = 1024
    grid = EA_PAD // Bb
    return pl.pallas_call(
        functools.partial(_passB_body, Bb),
        grid=(grid,),
        in_specs=[
            pl.BlockSpec((Bb, NODE_F), lambda i: (i, 2)),
            pl.BlockSpec((Bb, NODE_F), lambda i: (i, 1)),
            pl.BlockSpec((Bb, NODE_F), lambda i: (i, 1)),
            pl.BlockSpec((Bb, NODE_F), lambda i: (i, 0)),
            pl.BlockSpec((1, NODE_F), lambda i: (0, 0)),
            pl.BlockSpec((1, NODE_F), lambda i: (0, 0)),
            pl.BlockSpec((NODE_F, NODE_F), lambda i: (0, 0)),
            pl.BlockSpec((1, NODE_F), lambda i: (0, 0)),
            pl.BlockSpec((1, NODE_F), lambda i: (0, 0)),
        ],
        out_specs=[
            pl.BlockSpec((Bb, 128), lambda i: (i, 0)),
            pl.BlockSpec((Bb, 128), lambda i: (i, 0)),
        ],
        out_shape=[
            jax.ShapeDtypeStruct((EA_PAD, 128), F32),
            jax.ShapeDtypeStruct((EA_PAD, 128), F32),
        ],
        interpret=_INTERPRET,
    )(gTD, gTS, EF2, alpha, scale, shift, w1m, b0m, b1m)


def _cat_body(aggA_ref, aggB_ref, w_ref, bc_ref, o_ref, s1_ref, s2_ref):
    i = pl.program_id(0)
    agg = jnp.concatenate([aggA_ref[...], aggB_ref[...]], axis=1)
    o = jnp.dot(agg, w_ref[...], preferred_element_type=F32) + bc_ref[...]
    o_ref[...] = o

    @pl.when(i == 0)
    def _():
        s1_ref[...] = jnp.zeros_like(s1_ref)
        s2_ref[...] = jnp.zeros_like(s2_ref)

    s1_ref[...] += jnp.broadcast_to(jnp.sum(o, 0, keepdims=True), (8, NODE_F))
    s2_ref[...] += jnp.broadcast_to(jnp.sum(o * o, 0, keepdims=True),
                                    (8, NODE_F))


def _node_final_body(nf_ref, o_ref, sc_ref, sh_ref, out_ref):
    out_ref[...] = jax.nn.softplus(
        nf_ref[...] + o_ref[...] * sc_ref[...] + sh_ref[...])


def _agg_epilogue(aggA, aggB, nf, wcat, bcat, g, b):
    Bb = 1000
    o, s1, s2 = pl.pallas_call(
        _cat_body,
        grid=(N_NODES // Bb,),
        in_specs=[
            pl.BlockSpec((Bb, 128), lambda i: (i, 0)),
            pl.BlockSpec((Bb, 128), lambda i: (i, 0)),
            pl.BlockSpec((NODE_F, NODE_F), lambda i: (0, 0)),
            pl.BlockSpec((1, NODE_F), lambda i: (0, 0)),
        ],
        out_specs=[
            pl.BlockSpec((Bb, NODE_F), lambda i: (i, 0)),
            pl.BlockSpec((8, NODE_F), lambda i: (0, 0)),
            pl.BlockSpec((8, NODE_F), lambda i: (0, 0)),
        ],
        out_shape=[
            jax.ShapeDtypeStruct((N_NODES, NODE_F), F32),
            jax.ShapeDtypeStruct((8, NODE_F), F32),
            jax.ShapeDtypeStruct((8, NODE_F), F32),
        ],
        interpret=_INTERPRET,
    )(aggA, aggB, wcat, bcat)
    sc, sh = _finalize_bn(s1, s2, float(N_NODES), g, b)
    return pl.pallas_call(
        _node_final_body,
        grid=(N_NODES // Bb,),
        in_specs=[
            pl.BlockSpec((Bb, NODE_F), lambda i: (i, 0)),
            pl.BlockSpec((Bb, NODE_F), lambda i: (i, 0)),
            pl.BlockSpec((1, NODE_F), lambda i: (0, 0)),
            pl.BlockSpec((1, NODE_F), lambda i: (0, 0)),
        ],
        out_specs=pl.BlockSpec((Bb, NODE_F), lambda i: (i, 0)),
        out_shape=jax.ShapeDtypeStruct((N_NODES, NODE_F), F32),
        interpret=_INTERPRET,
    )(nf, o, sc, sh)


def _passAe_body(q_ref, kd_ref, len_ref, ang_ref, mk_ref, ak_ref, w1_ref,
                 b0_ref, bq_ref, b1_ref, a_ref, s1_ref, s2_ref):
    s = pl.program_id(0)
    j = pl.program_id(1)
    hk = jax.nn.silu(
        kd_ref[...]
        + jnp.dot(len_ref[...], mk_ref[0], preferred_element_type=F32)
        + jnp.dot(ang_ref[...], ak_ref[...], preferred_element_type=F32)
        + b0_ref[0])
    key = jnp.dot(hk, w1_ref[...], preferred_element_type=F32) + b1_ref[...]
    alpha = (q_ref[...] + bq_ref[...]) * key * INV_SQRT
    a_ref[...] = alpha[None]

    @pl.when((s == 0) & (j == 0))
    def _():
        s1_ref[...] = jnp.zeros_like(s1_ref)
        s2_ref[...] = jnp.zeros_like(s2_ref)

    s1_ref[...] += jnp.broadcast_to(jnp.sum(alpha, 0, keepdims=True),
                                    (8, NODE_F))
    s2_ref[...] += jnp.broadcast_to(jnp.sum(alpha * alpha, 0, keepdims=True),
                                    (8, NODE_F))


def _passA_edge(EP, len_flat, ang_flat, MkS, Ak, w1k, b0ks, bq, b1k):
    Bb = 640
    nj = E_REG // Bb
    return pl.pallas_call(
        _passAe_body,
        grid=(3, nj),
        in_specs=[
            pl.BlockSpec((Bb, NODE_F), lambda s, j: (j, 0)),
            pl.BlockSpec((Bb, NODE_F), lambda s, j: (j, 1)),
            pl.BlockSpec((Bb, NODE_F), lambda s, j: (s * nj + j, 0)),
            pl.BlockSpec((Bb, NODE_F), lambda s, j: (s * nj + j, 0)),
            pl.BlockSpec((1, NODE_F, NODE_F), lambda s, j: (s, 0, 0)),
            pl.BlockSpec((NODE_F, NODE_F), lambda s, j: (0, 0)),
            pl.BlockSpec((NODE_F, NODE_F), lambda s, j: (0, 0)),
            pl.BlockSpec((1, 1, NODE_F), lambda s, j: (s, 0, 0)),
            pl.BlockSpec((1, NODE_F), lambda s, j: (0, 0)),
            pl.BlockSpec((1, NODE_F), lambda s, j: (0, 0)),
        ],
        out_specs=[
            pl.BlockSpec((1, Bb, NODE_F), lambda s, j: (s, j, 0)),
            pl.BlockSpec((8, NODE_F), lambda s, j: (0, 0)),
            pl.BlockSpec((8, NODE_F), lambda s, j: (0, 0)),
        ],
        out_shape=[
            jax.ShapeDtypeStruct((3, E_REG, NODE_F), F32),
            jax.ShapeDtypeStruct((8, NODE_F), F32),
            jax.ShapeDtypeStruct((8, NODE_F), F32),
        ],
        interpret=_INTERPRET,
    )(EP, EP, len_flat, ang_flat, MkS, Ak, w1k, b0ks, bq, b1k)


def _passBe_body(vd_ref, len_ref, ang_ref, a_ref, sc_ref, sh_ref, mv_ref,
                 av_ref, w1_ref, b0_ref, b1_ref, og_ref, s1_ref, s2_ref):
    j = pl.program_id(0)
    s = pl.program_id(1)
    hm = jax.nn.silu(
        vd_ref[...]
        + jnp.dot(len_ref[...], mv_ref[0], preferred_element_type=F32)
        + jnp.dot(ang_ref[...], av_ref[...], preferred_element_type=F32)
        + b0_ref[0])
    msg = jnp.dot(hm, w1_ref[...], preferred_element_type=F32) + b1_ref[...]
    contrib = msg * jax.nn.sigmoid(a_ref[0] * sc_ref[...] + sh_ref[...])
    prev = jnp.where(s == 0, jnp.zeros_like(contrib), og_ref[...])
    total = prev + contrib
    og_ref[...] = total

    @pl.when((s == 2) & (j == 0))
    def _():
        s1_ref[...] = jnp.zeros_like(s1_ref)
        s2_ref[...] = jnp.zeros_like(s2_ref)

    @pl.when(s == 2)
    def _():
        s1_ref[...] += jnp.broadcast_to(jnp.sum(total, 0, keepdims=True),
                                        (8, NODE_F))
        s2_ref[...] += jnp.broadcast_to(
            jnp.sum(total * total, 0, keepdims=True), (8, NODE_F))


def _passB_edge(EP, len_flat, ang_flat, alpha_e, scale, shift, MvS, Av, w1m,
                b0ms, b1m):
    Bb = 640
    nj = E_REG // Bb
    return pl.pallas_call(
        _passBe_body,
        grid=(nj, 3),
        in_specs=[
            pl.BlockSpec((Bb, NODE_F), lambda j, s: (j, 2)),
            pl.BlockSpec((Bb, NODE_F), lambda j, s: (s * nj + j, 0)),
            pl.BlockSpec((Bb, NODE_F), lambda j, s: (s * nj + j, 0)),
            pl.BlockSpec((1, Bb, NODE_F), lambda j, s: (s, j, 0)),
            pl.BlockSpec((1, NODE_F), lambda j, s: (0, 0)),
            pl.BlockSpec((1, NODE_F), lambda j, s: (0, 0)),
            pl.BlockSpec((1, NODE_F, NODE_F), lambda j, s: (s, 0, 0)),
            pl.BlockSpec((NODE_F, NODE_F), lambda j, s: (0, 0)),
            pl.BlockSpec((NODE_F, NODE_F), lambda j, s: (0, 0)),
            pl.BlockSpec((1, 1, NODE_F), lambda j, s: (s, 0, 0)),
            pl.BlockSpec((1, NODE_F), lambda j, s: (0, 0)),
        ],
        out_specs=[
            pl.BlockSpec((Bb, NODE_F), lambda j, s: (j, 0)),
            pl.BlockSpec((8, NODE_F), lambda j, s: (0, 0)),
            pl.BlockSpec((8, NODE_F), lambda j, s: (0, 0)),
        ],
        out_shape=[
            jax.ShapeDtypeStruct((E_REG, NODE_F), F32),
            jax.ShapeDtypeStruct((8, NODE_F), F32),
            jax.ShapeDtypeStruct((8, NODE_F), F32),
        ],
        interpret=_INTERPRET,
    )(EP, len_flat, ang_flat, alpha_e, scale, shift, MvS, Av, w1m, b0ms, b1m)


def _edge_final_body(eg_ref, og_ref, sc_ref, sh_ref, o_ref):
    o_ref[...] = jax.nn.softplus(
        eg_ref[...] + og_ref[...] * sc_ref[...] + sh_ref[...])


def _edge_final(edge_features, outg, sc, sh):
    Bb = 640
    return pl.pallas_call(
        _edge_final_body,
        grid=(E_REG // Bb,),
        in_specs=[
            pl.BlockSpec((Bb, NODE_F), lambda j: (j, 0)),
            pl.BlockSpec((Bb, NODE_F), lambda j: (j, 0)),
            pl.BlockSpec((1, NODE_F), lambda j: (0, 0)),
            pl.BlockSpec((1, NODE_F), lambda j: (0, 0)),
        ],
        out_specs=pl.BlockSpec((Bb, NODE_F), lambda j: (j, 0)),
        out_shape=jax.ShapeDtypeStruct((E_REG, NODE_F), F32),
        interpret=_INTERPRET,
    )(edge_features, outg, sc, sh)


def _pool_body(nf_ref, batch_ref, wfc_ref, bfc_ref, wo_ref, bo_ref, o_ref):
    oh = (batch_ref[...] == lax.broadcasted_iota(jnp.int32, (N_GRAPHS, N_NODES),
                                                 0)).astype(F32)
    sums = jnp.dot(oh, nf_ref[...], preferred_element_type=F32)
    cnts = jnp.sum(oh, axis=1, keepdims=True)
    h = sums / jnp.maximum(cnts, 1.0)
    h = jax.nn.silu(jnp.dot(h, wfc_ref[...], preferred_element_type=F32)
                    + bfc_ref[...])
    logits = jnp.dot(h, wo_ref[...], preferred_element_type=F32) + bo_ref[...]
    z = logits - jnp.max(logits, axis=1, keepdims=True)
    lse = jnp.log(jnp.sum(jnp.exp(z), axis=1, keepdims=True))
    o_ref[...] = z - lse


def _pool(nf, batch_row, wfc, bfc, wop, bop):
    return pl.pallas_call(
        _pool_body,
        in_specs=[pl.BlockSpec(a.shape, lambda: tuple(0 for _ in a.shape))
                  for a in (nf, batch_row, wfc, bfc, wop, bop)],
        out_specs=pl.BlockSpec((N_GRAPHS, 128), lambda: (0, 0)),
        out_shape=jax.ShapeDtypeStruct((N_GRAPHS, 128), F32),
        interpret=_INTERPRET,
    )(nf, batch_row, wfc, bfc, wop, bop)


_SC_CH = 128
_B_PER_W = EA_PAD // 32
_N_CHUNK = _B_PER_W // _SC_CH


def _sc_gather_call(table, idx, width):
    mesh = plsc.VectorSubcoreMesh(core_axis_name="c", subcore_axis_name="s")

    @functools.partial(
        pl.kernel,
        out_type=jax.ShapeDtypeStruct((EA_PAD, width), F32),
        mesh=mesh,
        scratch_types=[
            pltpu.VMEM((_SC_CH,), jnp.int32),
            pltpu.VMEM((_SC_CH, width), F32),
            pltpu.SemaphoreType.DMA,
        ],
    )
    def gath(table_hbm, idx_hbm, out_hbm, idx_v, rows_v, sem):
        wid = lax.axis_index("s") * 2 + lax.axis_index("c")
        base = wid * _B_PER_W

        def body(gi, carry):
            off = base + gi * _SC_CH
            pltpu.sync_copy(idx_hbm.at[pl.ds(off, _SC_CH)], idx_v)
            pltpu.async_copy(table_hbm.at[idx_v], rows_v, sem).wait()
            pltpu.sync_copy(rows_v, out_hbm.at[pl.ds(off, _SC_CH)])
            return carry

        lax.fori_loop(0, _N_CHUNK, body, 0)

    return gath(table, idx)


N_PAD = 10240
_ROWS_PER_TILE = N_PAD // 16
_EDGES_PER_TILE = EA_PAD // 16
_N_SCHUNK = _EDGES_PER_TILE // _SC_CH


def _sc_scatter_call(mA, mB, dst, zer):
    mesh = plsc.VectorSubcoreMesh(core_axis_name="c", subcore_axis_name="s")

    @functools.partial(
        pl.kernel,
        out_type=jax.ShapeDtypeStruct((2, N_PAD, 128), F32),
        mesh=mesh,
        scratch_types=[
            pltpu.VMEM_SHARED((N_PAD, 128), F32),
            pltpu.VMEM((1, _SC_CH), jnp.int32),
            pltpu.VMEM((_SC_CH, 128), F32),
        ],
    )
    def scat(mA_hbm, mB_hbm, dst_hbm, zer_hbm, out_hbm, acc, idxb, rowb):
        c = lax.axis_index("c")
        s = lax.axis_index("s")
        pltpu.sync_copy(zer_hbm, acc.at[pl.ds(s * _ROWS_PER_TILE,
                                              _ROWS_PER_TILE)])
        plsc.subcore_barrier()
        base = s * _EDGES_PER_TILE

        def body(gi, carry):
            off = base + gi * _SC_CH
            pltpu.sync_copy(dst_hbm.at[pl.ds(off, _SC_CH)], idxb.at[0])

            @pl.when(c == 0)
            def _():
                pltpu.sync_copy(mA_hbm.at[pl.ds(off, _SC_CH)], rowb)

            @pl.when(c == 1)
            def _():
                pltpu.sync_copy(mB_hbm.at[pl.ds(off, _SC_CH)], rowb)

            pltpu.sync_copy(rowb, acc.at[idxb.at[0]], add=True)
            return carry

        lax.fori_loop(0, _N_SCHUNK, body, 0)
        plsc.subcore_barrier()
        pltpu.sync_copy(
            acc.at[pl.ds(s * _ROWS_PER_TILE, _ROWS_PER_TILE)],
            out_hbm.at[c, pl.ds(s * _ROWS_PER_TILE, _ROWS_PER_TILE)])

    return scat(mA, mB, dst, zer)


def _row(v):
    return v.reshape(1, -1)


def _finalize_bn(s1, s2, n, g, b):
    m = s1[0] / n
    var = s2[0] / n - m * m
    sc = g / jnp.sqrt(var + 1e-5)
    return _row(sc), _row(b - m * sc)


def kernel(x, edge_attr, edge_nei, params, wyckoff, inv_edge_type, edge_index,
           inv_edge_index, batch):
    p = params

    ei_src = jnp.concatenate([edge_index[0], inv_edge_index[0]])
    ei_dst = jnp.concatenate([edge_index[1], inv_edge_index[1]])
    padi = jnp.zeros((EA_PAD - EA,), ei_src.dtype)
    src_pad = jnp.concatenate([ei_src, padi]).astype(jnp.int32)
    dst_pad = jnp.concatenate([ei_dst, padi]).astype(jnp.int32)

    W = p['atom_emb']['w']
    wxp = jnp.zeros((128, NODE_F), F32).at[:x.shape[1]].set(W[:x.shape[1]])
    ewp = jnp.zeros((128, NODE_F), F32).at[:100].set(
        p['wyckoff_emb'] @ W[x.shape[1]:])
    xpad = jnp.pad(x, ((0, 0), (0, 128 - x.shape[1])))
    nf = _nf_embed(xpad, wyckoff.astype(jnp.int32).reshape(-1, 1), wxp, ewp,
                   _row(p['atom_emb']['b']))

    Wr = p['rbf']['w']
    W_bins, W_inv = Wr[:BINS], Wr[BINS:]
    br = _row(p['rbf']['b'])
    u_inv = _row((p['inv_edge_emb']['w'] @ W_inv)[0])
    cv_inv = _row(p['inv_edge_emb']['b'] @ W_inv)
    ef_d = -0.75 / jnp.linalg.norm(edge_attr, axis=1, keepdims=True)
    inv_d = inv_edge_type.astype(F32)[:, None]
    zpadd = jnp.zeros((EA_PAD - EA, 1), F32)
    d_edge = jnp.concatenate([ef_d, inv_d, zpadd], axis=0)
    g_edge = jnp.concatenate([jnp.zeros((E_REG, 1), F32),
                              jnp.ones((N_INV, 1), F32), zpadd], axis=0)
    edge_features = _rbf_block(d_edge, g_edge, W_bins, br, u_inv, cv_inv,
                               -4.0, 0.0, 640)

    nei_norm = jnp.linalg.norm(edge_nei, axis=-1)
    nei_len = -0.75 / nei_norm
    cosv = jnp.sum(edge_nei * edge_attr[:, None, :], axis=-1) / (
        nei_norm * jnp.linalg.norm(edge_attr, axis=1, keepdims=True))
    nei_angle = jnp.clip(cosv, -1.0, 1.0)
    d_len = nei_len.T.reshape(-1, 1)
    d_ang = nei_angle.T.reshape(-1, 1)
    zg = jnp.zeros_like(d_len)
    zu = jnp.zeros((1, NODE_F), F32)
    len_flat = _rbf_block(d_len, zg, W_bins, br, zu, zu, -4.0, 0.0, 640)
    ang_flat = _rbf_block(d_ang, zg, p['rbf_angle']['w'],
                          _row(p['rbf_angle']['b']), zu, zu, -1.0, 1.0, 640)

    zer = jnp.zeros((_ROWS_PER_TILE, 128), F32)

    for i in range(4):
        cp = p['convs'][i]
        K0 = cp['key_upd']['l0']['w']
        K0a, K0b, K0c = K0[:256], K0[256:512], K0[512:]
        M0 = cp['msg_upd']['l0']['w']
        M0a, M0b, M0c = M0[:256], M0[256:512], M0[512:]
        bk, bv, be = cp['k']['b'], cp['v']['b'], cp['e']['b']
        b0k = _row(cp['key_upd']['l0']['b'] + bk @ K0a + bk @ K0b + be @ K0c)
        b0m = _row(cp['msg_upd']['l0']['b'] + bv @ M0a + bv @ M0b + be @ M0c)
        WTD = jnp.concatenate(
            [cp['q']['w'], cp['k']['w'] @ K0a, cp['v']['w'] @ M0a], axis=1)
        WTS = jnp.concatenate([cp['k']['w'] @ K0b, cp['v']['w'] @ M0b], axis=1)
        WEF = jnp.concatenate([cp['e']['w'] @ K0c, cp['e']['w'] @ M0c], axis=1)

        TD = _mm(nf, WTD, 1000)
        TS = _mm(nf, WTS, 1000)
        EF2 = _mm(edge_features, WEF, 1024)
        gTD = _sc_gather_call(TD, dst_pad, 768)
        gTS = _sc_gather_call(TS, src_pad, 512)

        alpha, s1, s2 = _passA(gTD, gTS, EF2, cp['key_upd']['l1']['w'],
                               _row(cp['q']['b']), b0k,
                               _row(cp['key_upd']['l1']['b']))
        scale, shift = _finalize_bn(s1, s2, float(EA), cp['bn_att']['g'],
                                    cp['bn_att']['b'])
        mA, mB = _passB(gTD, gTS, EF2, alpha, scale, shift,
                        cp['msg_upd']['l1']['w'], b0m,
                        _row(cp['msg_upd']['l1']['b']))
        agg2 = _sc_scatter_call(mA, mB, dst_pad, zer)
        nf = _agg_epilogue(agg2[0], agg2[1], nf, cp['cat']['w'],
                           _row(cp['cat']['b']), _row(cp['bn']['g']),
                           _row(cp['bn']['b']))

        if i == 0:
            pe = p['edge_upd']
            K0 = pe['key_upd']['l0']['w']
            K0a, K0b, K0c = K0[:256], K0[256:512], K0[512:]
            M0 = pe['msg_upd']['l0']['w']
            M0a, M0b, M0c = M0[:256], M0[256:512], M0[512:]
            WEP = jnp.concatenate(
                [pe['q']['w'], pe['k']['w'] @ K0a, pe['v']['w'] @ M0a], axis=1)
            EP = _mm(edge_features, WEP, 640, rows_out=E_REG)
            MkS = jnp.stack([pe[f'k_e{s+1}']['w'] @ K0b for s in range(3)])
            MvS = jnp.stack([pe[f'v_e{s+1}']['w'] @ M0b for s in range(3)])
            b0ks = jnp.stack([
                (pe['key_upd']['l0']['b'] + pe['k']['b'] @ K0a
                 + pe[f'k_e{s+1}']['b'] @ K0b)[None] for s in range(3)])
            b0ms = jnp.stack([
                (pe['msg_upd']['l0']['b'] + pe['v']['b'] @ M0a
                 + pe[f'v_e{s+1}']['b'] @ M0b)[None] for s in range(3)])
            alpha_e, s1, s2 = _passA_edge(EP, len_flat, ang_flat, MkS, K0c,
                                          pe['key_upd']['l1']['w'], b0ks,
                                          _row(pe['q']['b']),
                                          _row(pe['key_upd']['l1']['b']))
            scale, shift = _finalize_bn(s1, s2, float(3 * E_REG),
                                        pe['bn_att']['g'], pe['bn_att']['b'])
            outg, s1, s2 = _passB_edge(EP, len_flat, ang_flat, alpha_e, scale,
                                       shift, MvS, M0c,
                                       pe['msg_upd']['l1']['w'], b0ms,
                                       _row(pe['msg_upd']['l1']['b']))
            sc2, sh2 = _finalize_bn(s1, s2, float(E_REG), pe['bn']['g'],
                                    pe['bn']['b'])
            upd = _edge_final(edge_features, outg, sc2, sh2)
            edge_features = jnp.concatenate([upd, edge_features[E_REG:]],
                                            axis=0)

    wop = jnp.zeros((NODE_F, 128), F32).at[:, :4].set(p['fc_out']['w'])
    bop = jnp.full((1, 128), -1e9, F32).at[0, :4].set(p['fc_out']['b'])
    res = _pool(nf, batch.astype(jnp.int32).reshape(1, -1), p['fc']['w'],
                _row(p['fc']['b']), wop, bop)
    return res[:, :4]


# speedup vs baseline: 1.4199x; 1.0299x over previous
import functools
import math

import jax
import jax.numpy as jnp
from jax import lax
from jax.experimental import pallas as pl
from jax.experimental.pallas import tpu as pltpu
from jax.experimental.pallas import tpu_sc as plsc

F32 = jnp.float32
N_NODES = 10000
E_REG = 160000
N_INV = 40000
EA = E_REG + N_INV
EA_PAD = 204800
BINS = 256
NODE_F = 256
N_GRAPHS = 64
INV_SQRT = 1.0 / math.sqrt(NODE_F)

_INTERPRET = False


def _rbf_body(gamma, vmin, step, d_ref, g_ref, w_ref, br_ref, u_ref, cv_ref,
              o_ref):
    d = d_ref[...]
    g = g_ref[...]
    centers = vmin + step * lax.broadcasted_iota(jnp.int32, (1, BINS),
                                                 1).astype(F32)
    feat = jnp.exp(-gamma * (d - centers) ** 2)
    acc = jnp.dot(feat, w_ref[...], preferred_element_type=F32)
    acc = acc + br_ref[...] + (d * g) * u_ref[...] + g * cv_ref[...]
    o_ref[...] = jax.nn.softplus(acc)


def _rbf_block(d, g, w, br, u, cv, vmin, vmax, Bb):
    R = d.shape[0]
    step = (vmax - vmin) / (BINS - 1)
    gamma = 1.0 / step
    return pl.pallas_call(
        functools.partial(_rbf_body, gamma, vmin, step),
        grid=(R // Bb,),
        in_specs=[
            pl.BlockSpec((Bb, 1), lambda i: (i, 0)),
            pl.BlockSpec((Bb, 1), lambda i: (i, 0)),
            pl.BlockSpec((BINS, NODE_F), lambda i: (0, 0)),
            pl.BlockSpec((1, NODE_F), lambda i: (0, 0)),
            pl.BlockSpec((1, NODE_F), lambda i: (0, 0)),
            pl.BlockSpec((1, NODE_F), lambda i: (0, 0)),
        ],
        out_specs=pl.BlockSpec((Bb, NODE_F), lambda i: (i, 0)),
        out_shape=jax.ShapeDtypeStruct((R, NODE_F), F32),
        interpret=_INTERPRET,
    )(d, g, w, br, u, cv)


def _mm_body(x_ref, w_ref, o_ref):
    o_ref[...] = jnp.dot(x_ref[...], w_ref[...], preferred_element_type=F32)


def _mm(x, w, Bb, rows_out=None):
    R = rows_out if rows_out is not None else x.shape[0]
    K = x.shape[1]
    Nw = w.shape[1]
    return pl.pallas_call(
        _mm_body,
        grid=(R // Bb,),
        in_specs=[
            pl.BlockSpec((Bb, K), lambda i: (i, 0)),
            pl.BlockSpec((K, Nw), lambda i: (0, 0)),
        ],
        out_specs=pl.BlockSpec((Bb, Nw), lambda i: (i, 0)),
        out_shape=jax.ShapeDtypeStruct((R, Nw), F32),
        interpret=_INTERPRET,
    )(x, w)


def _nf_embed_body(x_ref, wy_ref, wx_ref, ew_ref, b_ref, o_ref):
    oh = (wy_ref[...] == lax.broadcasted_iota(jnp.int32, (x_ref.shape[0], 128), 1)
          ).astype(F32)
    acc = jnp.dot(x_ref[...], wx_ref[...], preferred_element_type=F32)
    acc = acc + jnp.dot(oh, ew_ref[...], preferred_element_type=F32)
    o_ref[...] = acc + b_ref[...]


def _nf_embed(xpad, wy, wxp, ewp, b):
    Bb = 1000
    return pl.pallas_call(
        _nf_embed_body,
        grid=(N_NODES // Bb,),
        in_specs=[
            pl.BlockSpec((Bb, 128), lambda i: (i, 0)),
            pl.BlockSpec((Bb, 1), lambda i: (i, 0)),
            pl.BlockSpec((128, NODE_F), lambda i: (0, 0)),
            pl.BlockSpec((128, NODE_F), lambda i: (0, 0)),
            pl.BlockSpec((1, NODE_F), lambda i: (0, 0)),
        ],
        out_specs=pl.BlockSpec((Bb, NODE_F), lambda i: (i, 0)),
        out_shape=jax.ShapeDtypeStruct((N_NODES, NODE_F), F32),
        interpret=_INTERPRET,
    )(xpad, wy, wxp, ewp, b)


def _passA_body(Bb, qg_ref, kd_ref, ks_ref, ek_ref, w1_ref, bq_ref, b0_ref,
                b1_ref, a_ref, s1_ref, s2_ref):
    i = pl.program_id(0)
    hk = jax.nn.silu(kd_ref[...] + ks_ref[...] + ek_ref[...] + b0_ref[...])
    kj = jnp.dot(hk, w1_ref[...], preferred_element_type=F32) + b1_ref[...]
    alpha = (qg_ref[...] + bq_ref[...]) * kj * INV_SQRT
    a_ref[...] = alpha
    gidx = i * Bb + lax.broadcasted_iota(jnp.int32, (Bb, 1), 0)
    am = jnp.where(gidx < EA, alpha, 0.0)

    @pl.when(i == 0)
    def _():
        s1_ref[...] = jnp.zeros_like(s1_ref)
        s2_ref[...] = jnp.zeros_like(s2_ref)

    s1_ref[...] += jnp.broadcast_to(jnp.sum(am, 0, keepdims=True), (8, NODE_F))
    s2_ref[...] += jnp.broadcast_to(jnp.sum(am * am, 0, keepdims=True),
                                    (8, NODE_F))


def _passA(gTD, gTS, EF2, w1k, bq, b0k, b1k):
    Bb = 1024
    grid = EA_PAD // Bb
    return pl.pallas_call(
        functools.partial(_passA_body, Bb),
        grid=(grid,),
        in_specs=[
            pl.BlockSpec((Bb, NODE_F), lambda i: (i, 0)),
            pl.BlockSpec((Bb, NODE_F), lambda i: (i, 1)),
            pl.BlockSpec((Bb, NODE_F), lambda i: (i, 0)),
            pl.BlockSpec((Bb, NODE_F), lambda i: (i, 0)),
            pl.BlockSpec((NODE_F, NODE_F), lambda i: (0, 0)),
            pl.BlockSpec((1, NODE_F), lambda i: (0, 0)),
            pl.BlockSpec((1, NODE_F), lambda i: (0, 0)),
            pl.BlockSpec((1, NODE_F), lambda i: (0, 0)),
        ],
        out_specs=[
            pl.BlockSpec((Bb, NODE_F), lambda i: (i, 0)),
            pl.BlockSpec((8, NODE_F), lambda i: (0, 0)),
            pl.BlockSpec((8, NODE_F), lambda i: (0, 0)),
        ],
        out_shape=[
            jax.ShapeDtypeStruct((EA_PAD, NODE_F), F32),
            jax.ShapeDtypeStruct((8, NODE_F), F32),
            jax.ShapeDtypeStruct((8, NODE_F), F32),
        ],
        interpret=_INTERPRET,
    )(gTD, gTD, gTS, EF2, w1k, bq, b0k, b1k)


def _passB_body(Bb, vd_ref, vs_ref, em_ref, a_ref, sc_ref, sh_ref, w1_ref,
                b0_ref, b1_ref, ma_ref, mb_ref):
    i = pl.program_id(0)
    hm = jax.nn.silu(vd_ref[...] + vs_ref[...] + em_ref[...] + b0_ref[...])
    msg = jnp.dot(hm, w1_ref[...], preferred_element_type=F32) + b1_ref[...]
    gate = jax.nn.sigmoid(a_ref[...] * sc_ref[...] + sh_ref[...])
    mf = msg * gate
    gidx = i * Bb + lax.broadcasted_iota(jnp.int32, (Bb, 1), 0)
    mf = jnp.where(gidx < EA, mf, 0.0)
    ma_ref[...] = mf[:, :128]
    mb_ref[...] = mf[:, 128:]


def _passB(gTD, gTS, EF2, alpha, scale, shift, w1m, b0m, b1m):
    Bb = 1024
    grid = EA_PAD // Bb
    return pl.pallas_call(
        functools.partial(_passB_body, Bb),
        grid=(grid,),
        in_specs=[
            pl.BlockSpec((Bb, NODE_F), lambda i: (i, 2)),
            pl.BlockSpec((Bb, NODE_F), lambda i: (i, 1)),
            pl.BlockSpec((Bb, NODE_F), lambda i: (i, 1)),
            pl.BlockSpec((Bb, NODE_F), lambda i: (i, 0)),
            pl.BlockSpec((1, NODE_F), lambda i: (0, 0)),
            pl.BlockSpec((1, NODE_F), lambda i: (0, 0)),
            pl.BlockSpec((NODE_F, NODE_F), lambda i: (0, 0)),
            pl.BlockSpec((1, NODE_F), lambda i: (0, 0)),
            pl.BlockSpec((1, NODE_F), lambda i: (0, 0)),
        ],
        out_specs=[
            pl.BlockSpec((Bb, 128), lambda i: (i, 0)),
            pl.BlockSpec((Bb, 128), lambda i: (i, 0)),
        ],
        out_shape=[
            jax.ShapeDtypeStruct((EA_PAD, 128), F32),
            jax.ShapeDtypeStruct((EA_PAD, 128), F32),
        ],
        interpret=_INTERPRET,
    )(gTD, gTS, EF2, alpha, scale, shift, w1m, b0m, b1m)


def _cat_body(a0_ref, a1_ref, w_ref, bc_ref, o_ref, s1_ref, s2_ref):
    i = pl.program_id(0)
    agg = jnp.concatenate([a0_ref[...], a1_ref[...]], axis=1)
    o = jnp.dot(agg, w_ref[...], preferred_element_type=F32) + bc_ref[...]
    o_ref[...] = o

    @pl.when(i == 0)
    def _():
        s1_ref[...] = jnp.zeros_like(s1_ref)
        s2_ref[...] = jnp.zeros_like(s2_ref)

    s1_ref[...] += jnp.broadcast_to(jnp.sum(o, 0, keepdims=True), (8, NODE_F))
    s2_ref[...] += jnp.broadcast_to(jnp.sum(o * o, 0, keepdims=True),
                                    (8, NODE_F))


def _node_final_body(nf_ref, o_ref, sc_ref, sh_ref, out_ref):
    out_ref[...] = jax.nn.softplus(
        nf_ref[...] + o_ref[...] * sc_ref[...] + sh_ref[...])


def _agg_epilogue(agg4, nf, wcat, bcat, g, b):
    Bb = 1000
    o, s1, s2 = pl.pallas_call(
        _cat_body,
        grid=(N_NODES // Bb,),
        in_specs=[
            pl.BlockSpec((Bb, 128), lambda i: (i, 0)),
            pl.BlockSpec((Bb, 128), lambda i: (i, 0)),
            pl.BlockSpec((NODE_F, NODE_F), lambda i: (0, 0)),
            pl.BlockSpec((1, NODE_F), lambda i: (0, 0)),
        ],
        out_specs=[
            pl.BlockSpec((Bb, NODE_F), lambda i: (i, 0)),
            pl.BlockSpec((8, NODE_F), lambda i: (0, 0)),
            pl.BlockSpec((8, NODE_F), lambda i: (0, 0)),
        ],
        out_shape=[
            jax.ShapeDtypeStruct((N_NODES, NODE_F), F32),
            jax.ShapeDtypeStruct((8, NODE_F), F32),
            jax.ShapeDtypeStruct((8, NODE_F), F32),
        ],
        interpret=_INTERPRET,
    )(agg4[0], agg4[1], wcat, bcat)
    sc, sh = _finalize_bn(s1, s2, float(N_NODES), g, b)
    return pl.pallas_call(
        _node_final_body,
        grid=(N_NODES // Bb,),
        in_specs=[
            pl.BlockSpec((Bb, NODE_F), lambda i: (i, 0)),
            pl.BlockSpec((Bb, NODE_F), lambda i: (i, 0)),
            pl.BlockSpec((1, NODE_F), lambda i: (0, 0)),
            pl.BlockSpec((1, NODE_F), lambda i: (0, 0)),
        ],
        out_specs=pl.BlockSpec((Bb, NODE_F), lambda i: (i, 0)),
        out_shape=jax.ShapeDtypeStruct((N_NODES, NODE_F), F32),
        interpret=_INTERPRET,
    )(nf, o, sc, sh)


def _passAe_body(q_ref, kd_ref, len_ref, ang_ref, mk_ref, ak_ref, w1_ref,
                 b0_ref, bq_ref, b1_ref, a_ref, s1_ref, s2_ref):
    s = pl.program_id(0)
    j = pl.program_id(1)
    hk = jax.nn.silu(
        kd_ref[...]
        + jnp.dot(len_ref[...], mk_ref[0], preferred_element_type=F32)
        + jnp.dot(ang_ref[...], ak_ref[...], preferred_element_type=F32)
        + b0_ref[0])
    key = jnp.dot(hk, w1_ref[...], preferred_element_type=F32) + b1_ref[...]
    alpha = (q_ref[...] + bq_ref[...]) * key * INV_SQRT
    a_ref[...] = alpha[None]

    @pl.when((s == 0) & (j == 0))
    def _():
        s1_ref[...] = jnp.zeros_like(s1_ref)
        s2_ref[...] = jnp.zeros_like(s2_ref)

    s1_ref[...] += jnp.broadcast_to(jnp.sum(alpha, 0, keepdims=True),
                                    (8, NODE_F))
    s2_ref[...] += jnp.broadcast_to(jnp.sum(alpha * alpha, 0, keepdims=True),
                                    (8, NODE_F))


def _passA_edge(EP, len_flat, ang_flat, MkS, Ak, w1k, b0ks, bq, b1k):
    Bb = 640
    nj = E_REG // Bb
    return pl.pallas_call(
        _passAe_body,
        grid=(3, nj),
        in_specs=[
            pl.BlockSpec((Bb, NODE_F), lambda s, j: (j, 0)),
            pl.BlockSpec((Bb, NODE_F), lambda s, j: (j, 1)),
            pl.BlockSpec((Bb, NODE_F), lambda s, j: (s * nj + j, 0)),
            pl.BlockSpec((Bb, NODE_F), lambda s, j: (s * nj + j, 0)),
            pl.BlockSpec((1, NODE_F, NODE_F), lambda s, j: (s, 0, 0)),
            pl.BlockSpec((NODE_F, NODE_F), lambda s, j: (0, 0)),
            pl.BlockSpec((NODE_F, NODE_F), lambda s, j: (0, 0)),
            pl.BlockSpec((1, 1, NODE_F), lambda s, j: (s, 0, 0)),
            pl.BlockSpec((1, NODE_F), lambda s, j: (0, 0)),
            pl.BlockSpec((1, NODE_F), lambda s, j: (0, 0)),
        ],
        out_specs=[
            pl.BlockSpec((1, Bb, NODE_F), lambda s, j: (s, j, 0)),
            pl.BlockSpec((8, NODE_F), lambda s, j: (0, 0)),
            pl.BlockSpec((8, NODE_F), lambda s, j: (0, 0)),
        ],
        out_shape=[
            jax.ShapeDtypeStruct((3, E_REG, NODE_F), F32),
            jax.ShapeDtypeStruct((8, NODE_F), F32),
            jax.ShapeDtypeStruct((8, NODE_F), F32),
        ],
        interpret=_INTERPRET,
    )(EP, EP, len_flat, ang_flat, MkS, Ak, w1k, b0ks, bq, b1k)


def _passBe_body(vd_ref, len_ref, ang_ref, a_ref, sc_ref, sh_ref, mv_ref,
                 av_ref, w1_ref, b0_ref, b1_ref, og_ref, s1_ref, s2_ref):
    j = pl.program_id(0)
    s = pl.program_id(1)
    hm = jax.nn.silu(
        vd_ref[...]
        + jnp.dot(len_ref[...], mv_ref[0], preferred_element_type=F32)
        + jnp.dot(ang_ref[...], av_ref[...], preferred_element_type=F32)
        + b0_ref[0])
    msg = jnp.dot(hm, w1_ref[...], preferred_element_type=F32) + b1_ref[...]
    contrib = msg * jax.nn.sigmoid(a_ref[0] * sc_ref[...] + sh_ref[...])
    prev = jnp.where(s == 0, jnp.zeros_like(contrib), og_ref[...])
    total = prev + contrib
    og_ref[...] = total

    @pl.when((s == 2) & (j == 0))
    def _():
        s1_ref[...] = jnp.zeros_like(s1_ref)
        s2_ref[...] = jnp.zeros_like(s2_ref)

    @pl.when(s == 2)
    def _():
        s1_ref[...] += jnp.broadcast_to(jnp.sum(total, 0, keepdims=True),
                                        (8, NODE_F))
        s2_ref[...] += jnp.broadcast_to(
            jnp.sum(total * total, 0, keepdims=True), (8, NODE_F))


def _passB_edge(EP, len_flat, ang_flat, alpha_e, scale, shift, MvS, Av, w1m,
                b0ms, b1m):
    Bb = 640
    nj = E_REG // Bb
    return pl.pallas_call(
        _passBe_body,
        grid=(nj, 3),
        in_specs=[
            pl.BlockSpec((Bb, NODE_F), lambda j, s: (j, 2)),
            pl.BlockSpec((Bb, NODE_F), lambda j, s: (s * nj + j, 0)),
            pl.BlockSpec((Bb, NODE_F), lambda j, s: (s * nj + j, 0)),
            pl.BlockSpec((1, Bb, NODE_F), lambda j, s: (s, j, 0)),
            pl.BlockSpec((1, NODE_F), lambda j, s: (0, 0)),
            pl.BlockSpec((1, NODE_F), lambda j, s: (0, 0)),
            pl.BlockSpec((1, NODE_F, NODE_F), lambda j, s: (s, 0, 0)),
            pl.BlockSpec((NODE_F, NODE_F), lambda j, s: (0, 0)),
            pl.BlockSpec((NODE_F, NODE_F), lambda j, s: (0, 0)),
            pl.BlockSpec((1, 1, NODE_F), lambda j, s: (s, 0, 0)),
            pl.BlockSpec((1, NODE_F), lambda j, s: (0, 0)),
        ],
        out_specs=[
            pl.BlockSpec((Bb, NODE_F), lambda j, s: (j, 0)),
            pl.BlockSpec((8, NODE_F), lambda j, s: (0, 0)),
            pl.BlockSpec((8, NODE_F), lambda j, s: (0, 0)),
        ],
        out_shape=[
            jax.ShapeDtypeStruct((E_REG, NODE_F), F32),
            jax.ShapeDtypeStruct((8, NODE_F), F32),
            jax.ShapeDtypeStruct((8, NODE_F), F32),
        ],
        interpret=_INTERPRET,
    )(EP, len_flat, ang_flat, alpha_e, scale, shift, MvS, Av, w1m, b0ms, b1m)


def _edge_final_body(eg_ref, og_ref, sc_ref, sh_ref, o_ref):
    o_ref[...] = jax.nn.softplus(
        eg_ref[...] + og_ref[...] * sc_ref[...] + sh_ref[...])


def _edge_final(edge_features, outg, sc, sh):
    Bb = 640
    return pl.pallas_call(
        _edge_final_body,
        grid=(E_REG // Bb,),
        in_specs=[
            pl.BlockSpec((Bb, NODE_F), lambda j: (j, 0)),
            pl.BlockSpec((Bb, NODE_F), lambda j: (j, 0)),
            pl.BlockSpec((1, NODE_F), lambda j: (0, 0)),
            pl.BlockSpec((1, NODE_F), lambda j: (0, 0)),
        ],
        out_specs=pl.BlockSpec((Bb, NODE_F), lambda j: (j, 0)),
        out_shape=jax.ShapeDtypeStruct((E_REG, NODE_F), F32),
        interpret=_INTERPRET,
    )(edge_features, outg, sc, sh)


def _pool_body(nf_ref, batch_ref, wfc_ref, bfc_ref, wo_ref, bo_ref, o_ref):
    oh = (batch_ref[...] == lax.broadcasted_iota(jnp.int32, (N_GRAPHS, N_NODES),
                                                 0)).astype(F32)
    sums = jnp.dot(oh, nf_ref[...], preferred_element_type=F32)
    cnts = jnp.sum(oh, axis=1, keepdims=True)
    h = sums / jnp.maximum(cnts, 1.0)
    h = jax.nn.silu(jnp.dot(h, wfc_ref[...], preferred_element_type=F32)
                    + bfc_ref[...])
    logits = jnp.dot(h, wo_ref[...], preferred_element_type=F32) + bo_ref[...]
    z = logits - jnp.max(logits, axis=1, keepdims=True)
    lse = jnp.log(jnp.sum(jnp.exp(z), axis=1, keepdims=True))
    o_ref[...] = z - lse


def _pool(nf, batch_row, wfc, bfc, wop, bop):
    return pl.pallas_call(
        _pool_body,
        in_specs=[pl.BlockSpec(a.shape, lambda: tuple(0 for _ in a.shape))
                  for a in (nf, batch_row, wfc, bfc, wop, bop)],
        out_specs=pl.BlockSpec((N_GRAPHS, 128), lambda: (0, 0)),
        out_shape=jax.ShapeDtypeStruct((N_GRAPHS, 128), F32),
        interpret=_INTERPRET,
    )(nf, batch_row, wfc, bfc, wop, bop)


_SC_CH = 128
_B_PER_W = EA_PAD // 32
_N_CHUNK = _B_PER_W // _SC_CH


_G_CH = 64


def _sc_gather_call(table, idx, width):
    mesh = plsc.VectorSubcoreMesh(core_axis_name="c", subcore_axis_name="s")

    @functools.partial(
        pl.kernel,
        out_type=jax.ShapeDtypeStruct((EA_PAD, width), F32),
        mesh=mesh,
        scratch_types=[
            pltpu.VMEM((_B_PER_W // _G_CH, _G_CH), jnp.int32),
            pltpu.VMEM((_G_CH, width), F32),
            pltpu.VMEM((_G_CH, width), F32),
            pltpu.SemaphoreType.DMA,
            pltpu.SemaphoreType.DMA,
        ],
    )
    def gath(table_hbm, idx3_hbm, out_hbm, idx_v, rows0, rows1, sem0, sem1):
        wid = lax.axis_index("s") * 2 + lax.axis_index("c")
        base = wid * _B_PER_W
        pltpu.sync_copy(idx3_hbm.at[wid], idx_v)

        def body(i, carry):
            g0 = 2 * i
            g1 = g0 + 1
            h0 = pltpu.async_copy(table_hbm.at[idx_v.at[g0]], rows0, sem0)
            h0.wait()
            h1 = pltpu.async_copy(table_hbm.at[idx_v.at[g1]], rows1, sem1)
            pltpu.sync_copy(rows0,
                            out_hbm.at[pl.ds(base + g0 * _G_CH, _G_CH)])
            h1.wait()
            pltpu.sync_copy(rows1,
                            out_hbm.at[pl.ds(base + g1 * _G_CH, _G_CH)])
            return carry

        lax.fori_loop(0, _B_PER_W // (2 * _G_CH), body, 0)

    return gath(table, idx)


N_PAD = 10240
_ROWS_PER_TILE = N_PAD // 16
_EDGES_PER_TILE = EA_PAD // 16
_N_SCHUNK = _EDGES_PER_TILE // _SC_CH


def _sc_scatter_call(mA, mB, dst2, zer):
    mesh = plsc.VectorSubcoreMesh(core_axis_name="c", subcore_axis_name="s")

    @functools.partial(
        pl.kernel,
        out_type=jax.ShapeDtypeStruct((2, N_PAD, 128), F32),
        mesh=mesh,
        scratch_types=[
            pltpu.VMEM_SHARED((N_PAD, 128), F32),
            pltpu.VMEM((1, _SC_CH), jnp.int32),
            pltpu.VMEM((_SC_CH, 128), F32),
            pltpu.SemaphoreType.DMA,
        ],
    )
    def scat(mA_hbm, mB_hbm, dst2_hbm, zer_hbm, out_hbm, acc, idxb, rowb0,
             sem0):
        c = lax.axis_index("c")
        s = lax.axis_index("s")
        base = s * _EDGES_PER_TILE
        pltpu.sync_copy(zer_hbm, acc.at[pl.ds(s * _ROWS_PER_TILE,
                                              _ROWS_PER_TILE)])
        plsc.subcore_barrier()

        def body(g, carry):
            pltpu.sync_copy(dst2_hbm.at[s, pl.ds(g, 1)], idxb)

            @pl.when(c == 0)
            def _():
                pltpu.sync_copy(
                    mA_hbm.at[pl.ds(base + g * _SC_CH, _SC_CH)], rowb0)

            @pl.when(c == 1)
            def _():
                pltpu.sync_copy(
                    mB_hbm.at[pl.ds(base + g * _SC_CH, _SC_CH)], rowb0)

            pltpu.sync_copy(rowb0, acc.at[idxb.at[0]], add=True)
            return carry

        lax.fori_loop(0, _N_SCHUNK, body, 0)
        plsc.subcore_barrier()
        pltpu.sync_copy(
            acc.at[pl.ds(s * _ROWS_PER_TILE, _ROWS_PER_TILE)],
            out_hbm.at[c, pl.ds(s * _ROWS_PER_TILE, _ROWS_PER_TILE)])

    return scat(mA, mB, dst2, zer)


def _row(v):
    return v.reshape(1, -1)


def _finalize_bn(s1, s2, n, g, b):
    m = s1[0] / n
    var = s2[0] / n - m * m
    sc = g / jnp.sqrt(var + 1e-5)
    return _row(sc), _row(b - m * sc)


def kernel(x, edge_attr, edge_nei, params, wyckoff, inv_edge_type, edge_index,
           inv_edge_index, batch):
    p = params

    ei_src = jnp.concatenate([edge_index[0], inv_edge_index[0]])
    ei_dst = jnp.concatenate([edge_index[1], inv_edge_index[1]])
    padi = jnp.zeros((EA_PAD - EA,), ei_src.dtype)
    src_pad = jnp.concatenate([ei_src, padi]).astype(jnp.int32)
    dst_pad = jnp.concatenate([ei_dst, padi]).astype(jnp.int32)

    W = p['atom_emb']['w']
    wxp = jnp.zeros((128, NODE_F), F32).at[:x.shape[1]].set(W[:x.shape[1]])
    ewp = jnp.zeros((128, NODE_F), F32).at[:100].set(
        p['wyckoff_emb'] @ W[x.shape[1]:])
    xpad = jnp.pad(x, ((0, 0), (0, 128 - x.shape[1])))
    nf = _nf_embed(xpad, wyckoff.astype(jnp.int32).reshape(-1, 1), wxp, ewp,
                   _row(p['atom_emb']['b']))

    Wr = p['rbf']['w']
    W_bins, W_inv = Wr[:BINS], Wr[BINS:]
    br = _row(p['rbf']['b'])
    u_inv = _row((p['inv_edge_emb']['w'] @ W_inv)[0])
    cv_inv = _row(p['inv_edge_emb']['b'] @ W_inv)
    ef_d = -0.75 / jnp.linalg.norm(edge_attr, axis=1, keepdims=True)
    inv_d = inv_edge_type.astype(F32)[:, None]
    zpadd = jnp.zeros((EA_PAD - EA, 1), F32)
    d_edge = jnp.concatenate([ef_d, inv_d, zpadd], axis=0)
    g_edge = jnp.concatenate([jnp.zeros((E_REG, 1), F32),
                              jnp.ones((N_INV, 1), F32), zpadd], axis=0)
    edge_features = _rbf_block(d_edge, g_edge, W_bins, br, u_inv, cv_inv,
                               -4.0, 0.0, 640)

    nei_norm = jnp.linalg.norm(edge_nei, axis=-1)
    nei_len = -0.75 / nei_norm
    cosv = jnp.sum(edge_nei * edge_attr[:, None, :], axis=-1) / (
        nei_norm * jnp.linalg.norm(edge_attr, axis=1, keepdims=True))
    nei_angle = jnp.clip(cosv, -1.0, 1.0)
    d_len = nei_len.T.reshape(-1, 1)
    d_ang = nei_angle.T.reshape(-1, 1)
    zg = jnp.zeros_like(d_len)
    zu = jnp.zeros((1, NODE_F), F32)
    len_flat = _rbf_block(d_len, zg, W_bins, br, zu, zu, -4.0, 0.0, 640)
    ang_flat = _rbf_block(d_ang, zg, p['rbf_angle']['w'],
                          _row(p['rbf_angle']['b']), zu, zu, -1.0, 1.0, 640)

    zer = jnp.zeros((_ROWS_PER_TILE, 128), F32)
    dst2_pad = dst_pad.reshape(16, _N_SCHUNK, _SC_CH)
    dst3 = dst_pad.reshape(32, _B_PER_W // _G_CH, _G_CH)
    src3 = src_pad.reshape(32, _B_PER_W // _G_CH, _G_CH)

    for i in range(4):
        cp = p['convs'][i]
        K0 = cp['key_upd']['l0']['w']
        K0a, K0b, K0c = K0[:256], K0[256:512], K0[512:]
        M0 = cp['msg_upd']['l0']['w']
        M0a, M0b, M0c = M0[:256], M0[256:512], M0[512:]
        bk, bv, be = cp['k']['b'], cp['v']['b'], cp['e']['b']
        b0k = _row(cp['key_upd']['l0']['b'] + bk @ K0a + bk @ K0b + be @ K0c)
        b0m = _row(cp['msg_upd']['l0']['b'] + bv @ M0a + bv @ M0b + be @ M0c)
        WTD = jnp.concatenate(
            [cp['q']['w'], cp['k']['w'] @ K0a, cp['v']['w'] @ M0a], axis=1)
        WTS = jnp.concatenate([cp['k']['w'] @ K0b, cp['v']['w'] @ M0b], axis=1)
        WEF = jnp.concatenate([cp['e']['w'] @ K0c, cp['e']['w'] @ M0c], axis=1)

        TD = _mm(nf, WTD, 1000)
        TS = _mm(nf, WTS, 1000)
        EF2 = _mm(edge_features, WEF, 1024)
        gTD = _sc_gather_call(TD, dst3, 768)
        gTS = _sc_gather_call(TS, src3, 512)

        alpha, s1, s2 = _passA(gTD, gTS, EF2, cp['key_upd']['l1']['w'],
                               _row(cp['q']['b']), b0k,
                               _row(cp['key_upd']['l1']['b']))
        scale, shift = _finalize_bn(s1, s2, float(EA), cp['bn_att']['g'],
                                    cp['bn_att']['b'])
        mA, mB = _passB(gTD, gTS, EF2, alpha, scale, shift,
                        cp['msg_upd']['l1']['w'], b0m,
                        _row(cp['msg_upd']['l1']['b']))
        agg4 = _sc_scatter_call(mA, mB, dst2_pad, zer)
        nf = _agg_epilogue(agg4, nf, cp['cat']['w'],
                           _row(cp['cat']['b']), _row(cp['bn']['g']),
                           _row(cp['bn']['b']))

        if i == 0:
            pe = p['edge_upd']
            K0 = pe['key_upd']['l0']['w']
            K0a, K0b, K0c = K0[:256], K0[256:512], K0[512:]
            M0 = pe['msg_upd']['l0']['w']
            M0a, M0b, M0c = M0[:256], M0[256:512], M0[512:]
            WEP = jnp.concatenate(
                [pe['q']['w'], pe['k']['w'] @ K0a, pe['v']['w'] @ M0a], axis=1)
            EP = _mm(edge_features, WEP, 640, rows_out=E_REG)
            MkS = jnp.stack([pe[f'k_e{s+1}']['w'] @ K0b for s in range(3)])
            MvS = jnp.stack([pe[f'v_e{s+1}']['w'] @ M0b for s in range(3)])
            b0ks = jnp.stack([
                (pe['key_upd']['l0']['b'] + pe['k']['b'] @ K0a
                 + pe[f'k_e{s+1}']['b'] @ K0b)[None] for s in range(3)])
            b0ms = jnp.stack([
                (pe['msg_upd']['l0']['b'] + pe['v']['b'] @ M0a
                 + pe[f'v_e{s+1}']['b'] @ M0b)[None] for s in range(3)])
            alpha_e, s1, s2 = _passA_edge(EP, len_flat, ang_flat, MkS, K0c,
                                          pe['key_upd']['l1']['w'], b0ks,
                                          _row(pe['q']['b']),
                                          _row(pe['key_upd']['l1']['b']))
            scale, shift = _finalize_bn(s1, s2, float(3 * E_REG),
                                        pe['bn_att']['g'], pe['bn_att']['b'])
            outg, s1, s2 = _passB_edge(EP, len_flat, ang_flat, alpha_e, scale,
                                       shift, MvS, M0c,
                                       pe['msg_upd']['l1']['w'], b0ms,
                                       _row(pe['msg_upd']['l1']['b']))
            sc2, sh2 = _finalize_bn(s1, s2, float(E_REG), pe['bn']['g'],
                                    pe['bn']['b'])
            upd = _edge_final(edge_features, outg, sc2, sh2)
            edge_features = jnp.concatenate([upd, edge_features[E_REG:]],
                                            axis=0)

    wop = jnp.zeros((NODE_F, 128), F32).at[:, :4].set(p['fc_out']['w'])
    bop = jnp.full((1, 128), -1e9, F32).at[0, :4].set(p['fc_out']['b'])
    res = _pool(nf, batch.astype(jnp.int32).reshape(1, -1), p['fc']['w'],
                _row(p['fc']['b']), wop, bop)
    return res[:, :4]
